# baseline (speedup 1.0000x reference)
import jax
import jax.numpy as jnp
from jax import lax
from jax.experimental import pallas as pl
from jax.experimental.pallas import tpu as pltpu
from jax.experimental.pallas import tpu_sc as plsc

N = 10000
E = 160000
D_IN = 256
D_OUT = 256
H = 128
NP = 10240
STRIPE = NP // 16
CH = 128
KTOT = E // CH

_MESH = plsc.VectorSubcoreMesh(core_axis_name="c", subcore_axis_name="s")

def _z16():
    return jnp.zeros((16,), jnp.float32)


def _o16():
    return jnp.ones((16,), jnp.float32)


RING_A = 8
WIN_A = 4


def _hist_body(ei_hbm, p0_hbm, p1_hbm, ridx_v, ones_v, zv, isem, ssem, table_sh):
    c = lax.axis_index("c")
    s = lax.axis_index("s")
    wid = c * 16 + s
    nk = 39 + (wid < 2)

    def zrow(j, carry):
        zv[pl.ds(j * 16, 16)] = _z16()
        return carry

    lax.fori_loop(0, STRIPE // 16, zrow, 0)
    for j in range(CH // 16):
        ones_v[pl.ds(j * 16, 16)] = _o16()
    pltpu.sync_copy(zv, table_sh.at[pl.ds(s * STRIPE, STRIPE)])
    plsc.subcore_barrier()

    def idx_load(i):
        pltpu.async_copy(ei_hbm.at[0, pl.ds((wid + 32 * i) * CH, CH)],
                         ridx_v.at[i % RING_A], isem)

    def idx_wait(i):
        pltpu.make_async_copy(ei_hbm.at[0, pl.ds((wid + 32 * i) * CH, CH)],
                              ridx_v.at[i % RING_A], isem).wait()

    def sc_desc(i):
        return pltpu.make_async_copy(ones_v, table_sh.at[ridx_v.at[i % RING_A]],
                                     ssem)

    for i in range(WIN_A):
        idx_load(i)

    def chunk(i, carry):
        @pl.when(i >= WIN_A)
        def _():
            sc_desc(i - WIN_A).wait()

        @pl.when(i + WIN_A < nk)
        def _():
            idx_load(i + WIN_A)

        idx_wait(i)
        pltpu.async_copy(ones_v, table_sh.at[ridx_v.at[i % RING_A]], ssem,
                         add=True)
        return carry

    lax.fori_loop(0, nk, chunk, 0)

    def drain(j, carry):
        sc_desc(nk - WIN_A + j).wait()
        return carry

    lax.fori_loop(0, WIN_A, drain, 0)
    plsc.subcore_barrier()

    @pl.when(c == 0)
    def _():
        pltpu.sync_copy(table_sh.at[pl.ds(s * STRIPE, STRIPE)],
                        p0_hbm.at[pl.ds(s * STRIPE, STRIPE)])

    @pl.when(c == 1)
    def _():
        pltpu.sync_copy(table_sh.at[pl.ds(s * STRIPE, STRIPE)],
                        p1_hbm.at[pl.ds(s * STRIPE, STRIPE)])


_hist = pl.kernel(
    _hist_body,
    out_type=[
        jax.ShapeDtypeStruct((NP,), jnp.float32),
        jax.ShapeDtypeStruct((NP,), jnp.float32),
    ],
    mesh=_MESH,
    scratch_types=[
        pltpu.VMEM((RING_A, CH), jnp.int32),
        pltpu.VMEM((CH,), jnp.float32),
        pltpu.VMEM((STRIPE,), jnp.float32),
        pltpu.SemaphoreType.DMA,
        pltpu.SemaphoreType.DMA,
        pltpu.VMEM_SHARED((NP,), jnp.float32),
    ],
)


_RB = 1000


def _mm_body(x_ref, w_ref, xw_ref):
    xw_ref[...] = jnp.dot(x_ref[...], w_ref[...],
                          preferred_element_type=jnp.float32)


def _matmul(x, w):
    grid = N // _RB
    return pl.pallas_call(
        _mm_body,
        grid=(grid,),
        in_specs=[
            pl.BlockSpec((_RB, D_IN), lambda i: (i, 0)),
            pl.BlockSpec((D_IN, D_OUT), lambda i: (0, 0)),
        ],
        out_specs=pl.BlockSpec((_RB, D_OUT), lambda i: (i, 0)),
        out_shape=jax.ShapeDtypeStruct((N, D_OUT), jnp.float32),
    )(x, w)


def _yscale_body(xw_ref, deg_ref, y0_ref, y1_ref, dis_ref):
    deg = deg_ref[...]
    dis = jnp.where(deg > 0.0, lax.rsqrt(deg), 0.0)
    y = xw_ref[...] * dis
    y0_ref[...] = y[:, :H]
    y1_ref[...] = y[:, H:]
    dis_ref[...] = dis


def _yscale(xw, deg):
    grid = N // _RB
    return pl.pallas_call(
        _yscale_body,
        grid=(grid,),
        in_specs=[
            pl.BlockSpec((_RB, D_OUT), lambda i: (i, 0)),
            pl.BlockSpec((_RB, 1), lambda i: (i, 0)),
        ],
        out_specs=[
            pl.BlockSpec((_RB, H), lambda i: (i, 0)),
            pl.BlockSpec((_RB, H), lambda i: (i, 0)),
            pl.BlockSpec((_RB, 1), lambda i: (i, 0)),
        ],
        out_shape=[
            jax.ShapeDtypeStruct((N, H), jnp.float32),
            jax.ShapeDtypeStruct((N, H), jnp.float32),
            jax.ShapeDtypeStruct((N, 1), jnp.float32),
        ],
    )(xw, deg)


RING_I = 4
RING_R = 3
NZCH = N // CH
ZREM = N % CH


def _acc_blocks(s, copy_block):
    for jj in range(5):
        j = s + 16 * jj

        @pl.when(j < NZCH)
        def _():
            copy_block(j * CH, CH)

        @pl.when(j == NZCH)
        def _():
            copy_block(NZCH * CH, ZREM)


def _scatter_body(y0_hbm, y1_hbm, ei_hbm,
                  o0_hbm, o1_hbm, cidx_v, ridx_v, rows_v,
                  isem, gsem, ssem, acc_sh):
    c = lax.axis_index("c")
    s = lax.axis_index("s")
    nk = 78 + (s < 2)

    def zrow(j, carry):
        for q in range(H // 16):
            rows_v[0, j, pl.ds(q * 16, 16)] = _z16()
        return carry

    lax.fori_loop(0, CH, zrow, 0)

    def zblock(off, nrows):
        pltpu.sync_copy(rows_v.at[0, pl.ds(0, nrows)],
                        acc_sh.at[pl.ds(off, nrows)])

    _acc_blocks(s, zblock)
    plsc.subcore_barrier()

    def idx_load(i):
        e0 = (s + 16 * i) * CH
        pltpu.async_copy(ei_hbm.at[1, pl.ds(e0, CH)], cidx_v.at[i % RING_I], isem)
        pltpu.async_copy(ei_hbm.at[0, pl.ds(e0, CH)], ridx_v.at[i % RING_I], isem)

    def idx_wait(i):
        e0 = (s + 16 * i) * CH
        pltpu.make_async_copy(ei_hbm.at[1, pl.ds(e0, CH)],
                              cidx_v.at[i % RING_I], isem).wait()
        pltpu.make_async_copy(ei_hbm.at[0, pl.ds(e0, CH)],
                              ridx_v.at[i % RING_I], isem).wait()

    def run(y_hbm):
        def gather_go(i):
            pltpu.async_copy(y_hbm.at[cidx_v.at[i % RING_I]],
                             rows_v.at[i % RING_R], gsem)

        def gather_desc(i):
            return pltpu.make_async_copy(y_hbm.at[cidx_v.at[i % RING_I]],
                                         rows_v.at[i % RING_R], gsem)

        def scatter_desc(i):
            return pltpu.make_async_copy(rows_v.at[i % RING_R],
                                         acc_sh.at[ridx_v.at[i % RING_I]], ssem)

        for i in range(3):
            idx_load(i)
        for i in range(2):
            idx_wait(i)
            gather_go(i)

        def chunk(i, carry):
            gather_desc(i).wait()

            @pl.when(i >= 1)
            def _():
                scatter_desc(i - 1).wait()

            @pl.when(i + 3 < nk)
            def _():
                idx_load(i + 3)

            @pl.when(i + 2 < nk)
            def _():
                idx_wait(i + 2)
                gather_go(i + 2)

            pltpu.async_copy(rows_v.at[i % RING_R],
                             acc_sh.at[ridx_v.at[i % RING_I]], ssem, add=True)
            return carry

        lax.fori_loop(0, nk, chunk, 0)
        scatter_desc(nk - 1).wait()

    @pl.when(c == 0)
    def _():
        run(y0_hbm)

    @pl.when(c == 1)
    def _():
        run(y1_hbm)

    plsc.subcore_barrier()

    @pl.when(c == 0)
    def _():
        def dblock(off, nrows):
            pltpu.sync_copy(acc_sh.at[pl.ds(off, nrows)],
                            o0_hbm.at[pl.ds(off, nrows)])

        _acc_blocks(s, dblock)

    @pl.when(c == 1)
    def _():
        def dblock(off, nrows):
            pltpu.sync_copy(acc_sh.at[pl.ds(off, nrows)],
                            o1_hbm.at[pl.ds(off, nrows)])

        _acc_blocks(s, dblock)


_scatter = pl.kernel(
    _scatter_body,
    out_type=[
        jax.ShapeDtypeStruct((N, H), jnp.float32),
        jax.ShapeDtypeStruct((N, H), jnp.float32),
    ],
    mesh=_MESH,
    scratch_types=[
        pltpu.VMEM((RING_I, CH), jnp.int32),
        pltpu.VMEM((RING_I, CH), jnp.int32),
        pltpu.VMEM((RING_R, CH, H), jnp.float32),
        pltpu.SemaphoreType.DMA,
        pltpu.SemaphoreType.DMA,
        pltpu.SemaphoreType.DMA,
        pltpu.VMEM_SHARED((N, H), jnp.float32),
    ],
)


def _scale_body(o0_ref, o1_ref, dis_ref, out_ref):
    dis = dis_ref[...]
    out_ref[:, :H] = o0_ref[...] * dis
    out_ref[:, H:] = o1_ref[...] * dis


def _scale(o0, o1, dis):
    grid = N // _RB
    return pl.pallas_call(
        _scale_body,
        grid=(grid,),
        in_specs=[
            pl.BlockSpec((_RB, H), lambda i: (i, 0)),
            pl.BlockSpec((_RB, H), lambda i: (i, 0)),
            pl.BlockSpec((_RB, 1), lambda i: (i, 0)),
        ],
        out_specs=pl.BlockSpec((_RB, D_OUT), lambda i: (i, 0)),
        out_shape=jax.ShapeDtypeStruct((N, D_OUT), jnp.float32),
    )(o0, o1, dis)


def kernel(x, edge_index, W):
    ei = edge_index.astype(jnp.int32)
    xw = _matmul(x, W)
    p0, p1 = _hist(ei)
    y0, y1, dis = _yscale(xw, (p0 + p1).reshape(NP, 1))
    o0, o1 = _scatter(y0, y1, ei)
    return _scale(o0, o1, dis)


# speedup vs baseline: 1.0092x; 1.0092x over previous
import jax
import jax.numpy as jnp
from jax import lax
from jax.experimental import pallas as pl
from jax.experimental.pallas import tpu as pltpu
from jax.experimental.pallas import tpu_sc as plsc

N = 10000
E = 160000
D_IN = 256
D_OUT = 256
H = 128
NP = 10240
STRIPE = NP // 16
CH = 128
KTOT = E // CH

_MESH = plsc.VectorSubcoreMesh(core_axis_name="c", subcore_axis_name="s")

def _z16():
    return jnp.zeros((16,), jnp.float32)


def _o16():
    return jnp.ones((16,), jnp.float32)


RING_A = 8
WIN_A = 4


def _hist_body(ei_hbm, p0_hbm, p1_hbm, ridx_v, ones_v, zv, isem, ssem, table_sh):
    c = lax.axis_index("c")
    s = lax.axis_index("s")
    wid = c * 16 + s
    nk = 39 + (wid < 2)

    def zrow(j, carry):
        zv[pl.ds(j * 16, 16)] = _z16()
        return carry

    lax.fori_loop(0, STRIPE // 16, zrow, 0)
    for j in range(CH // 16):
        ones_v[pl.ds(j * 16, 16)] = _o16()
    pltpu.sync_copy(zv, table_sh.at[pl.ds(s * STRIPE, STRIPE)])
    plsc.subcore_barrier()

    def idx_load(i):
        pltpu.async_copy(ei_hbm.at[0, pl.ds((wid + 32 * i) * CH, CH)],
                         ridx_v.at[i % RING_A], isem)

    def idx_wait(i):
        pltpu.make_async_copy(ei_hbm.at[0, pl.ds((wid + 32 * i) * CH, CH)],
                              ridx_v.at[i % RING_A], isem).wait()

    def sc_desc(i):
        return pltpu.make_async_copy(ones_v, table_sh.at[ridx_v.at[i % RING_A]],
                                     ssem)

    for i in range(WIN_A):
        idx_load(i)

    def chunk(i, carry):
        @pl.when(i >= WIN_A)
        def _():
            sc_desc(i - WIN_A).wait()

        @pl.when(i + WIN_A < nk)
        def _():
            idx_load(i + WIN_A)

        idx_wait(i)
        pltpu.async_copy(ones_v, table_sh.at[ridx_v.at[i % RING_A]], ssem,
                         add=True)
        return carry

    lax.fori_loop(0, nk, chunk, 0)

    def drain(j, carry):
        sc_desc(nk - WIN_A + j).wait()
        return carry

    lax.fori_loop(0, WIN_A, drain, 0)
    plsc.subcore_barrier()

    @pl.when(c == 0)
    def _():
        pltpu.sync_copy(table_sh.at[pl.ds(s * STRIPE, STRIPE)],
                        p0_hbm.at[pl.ds(s * STRIPE, STRIPE)])

    @pl.when(c == 1)
    def _():
        pltpu.sync_copy(table_sh.at[pl.ds(s * STRIPE, STRIPE)],
                        p1_hbm.at[pl.ds(s * STRIPE, STRIPE)])


_hist = pl.kernel(
    _hist_body,
    out_type=[
        jax.ShapeDtypeStruct((NP,), jnp.float32),
        jax.ShapeDtypeStruct((NP,), jnp.float32),
    ],
    mesh=_MESH,
    scratch_types=[
        pltpu.VMEM((RING_A, CH), jnp.int32),
        pltpu.VMEM((CH,), jnp.float32),
        pltpu.VMEM((STRIPE,), jnp.float32),
        pltpu.SemaphoreType.DMA,
        pltpu.SemaphoreType.DMA,
        pltpu.VMEM_SHARED((NP,), jnp.float32),
    ],
)


_RB = 1000


def _mm_body(x_ref, w_ref, xw_ref):
    xw_ref[...] = jnp.dot(x_ref[...], w_ref[...],
                          preferred_element_type=jnp.float32)


def _matmul(x, w):
    grid = N // _RB
    return pl.pallas_call(
        _mm_body,
        grid=(grid,),
        in_specs=[
            pl.BlockSpec((_RB, D_IN), lambda i: (i, 0)),
            pl.BlockSpec((D_IN, D_OUT), lambda i: (0, 0)),
        ],
        out_specs=pl.BlockSpec((_RB, D_OUT), lambda i: (i, 0)),
        out_shape=jax.ShapeDtypeStruct((N, D_OUT), jnp.float32),
    )(x, w)


def _yscale_body(xw_ref, deg_ref, y0_ref, y1_ref):
    deg = deg_ref[...]
    dis = jnp.where(deg > 0.0, lax.rsqrt(deg), 0.0)
    y = xw_ref[...] * dis
    y0_ref[...] = y[:, :H]
    y1_ref[...] = y[:, H:]


def _yscale(xw, deg):
    grid = N // _RB
    return pl.pallas_call(
        _yscale_body,
        grid=(grid,),
        in_specs=[
            pl.BlockSpec((_RB, D_OUT), lambda i: (i, 0)),
            pl.BlockSpec((_RB, 1), lambda i: (i, 0)),
        ],
        out_specs=[
            pl.BlockSpec((_RB, H), lambda i: (i, 0)),
            pl.BlockSpec((_RB, H), lambda i: (i, 0)),
        ],
        out_shape=[
            jax.ShapeDtypeStruct((N, H), jnp.float32),
            jax.ShapeDtypeStruct((N, H), jnp.float32),
        ],
    )(xw, deg)


RING_I = 4
RING_R = 3
NZCH = N // CH
ZREM = N % CH


def _acc_blocks(s, copy_block):
    for jj in range(5):
        j = s + 16 * jj

        @pl.when(j < NZCH)
        def _():
            copy_block(j * CH, CH)

        @pl.when(j == NZCH)
        def _():
            copy_block(NZCH * CH, ZREM)


def _scatter_body(y0_hbm, y1_hbm, ei_hbm,
                  o0_hbm, o1_hbm, cidx_v, ridx_v, rows_v,
                  isem, gsem, ssem, acc_sh):
    c = lax.axis_index("c")
    s = lax.axis_index("s")
    nk = 78 + (s < 2)

    def zrow(j, carry):
        for q in range(H // 16):
            rows_v[0, j, pl.ds(q * 16, 16)] = _z16()
        return carry

    lax.fori_loop(0, CH, zrow, 0)

    def zblock(off, nrows):
        pltpu.sync_copy(rows_v.at[0, pl.ds(0, nrows)],
                        acc_sh.at[pl.ds(off, nrows)])

    _acc_blocks(s, zblock)
    plsc.subcore_barrier()

    def idx_load(i):
        e0 = (s + 16 * i) * CH
        pltpu.async_copy(ei_hbm.at[1, pl.ds(e0, CH)], cidx_v.at[i % RING_I], isem)
        pltpu.async_copy(ei_hbm.at[0, pl.ds(e0, CH)], ridx_v.at[i % RING_I], isem)

    def idx_wait(i):
        e0 = (s + 16 * i) * CH
        pltpu.make_async_copy(ei_hbm.at[1, pl.ds(e0, CH)],
                              cidx_v.at[i % RING_I], isem).wait()
        pltpu.make_async_copy(ei_hbm.at[0, pl.ds(e0, CH)],
                              ridx_v.at[i % RING_I], isem).wait()

    def run(y_hbm):
        def gather_go(i):
            pltpu.async_copy(y_hbm.at[cidx_v.at[i % RING_I]],
                             rows_v.at[i % RING_R], gsem)

        def gather_desc(i):
            return pltpu.make_async_copy(y_hbm.at[cidx_v.at[i % RING_I]],
                                         rows_v.at[i % RING_R], gsem)

        def scatter_desc(i):
            return pltpu.make_async_copy(rows_v.at[i % RING_R],
                                         acc_sh.at[ridx_v.at[i % RING_I]], ssem)

        for i in range(3):
            idx_load(i)
        for i in range(2):
            idx_wait(i)
            gather_go(i)

        def chunk(i, carry):
            gather_desc(i).wait()

            @pl.when(i >= 1)
            def _():
                scatter_desc(i - 1).wait()

            @pl.when(i + 3 < nk)
            def _():
                idx_load(i + 3)

            @pl.when(i + 2 < nk)
            def _():
                idx_wait(i + 2)
                gather_go(i + 2)

            pltpu.async_copy(rows_v.at[i % RING_R],
                             acc_sh.at[ridx_v.at[i % RING_I]], ssem, add=True)
            return carry

        lax.fori_loop(0, nk, chunk, 0)
        scatter_desc(nk - 1).wait()

    @pl.when(c == 0)
    def _():
        run(y0_hbm)

    @pl.when(c == 1)
    def _():
        run(y1_hbm)

    plsc.subcore_barrier()

    @pl.when(c == 0)
    def _():
        def dblock(off, nrows):
            pltpu.sync_copy(acc_sh.at[pl.ds(off, nrows)],
                            o0_hbm.at[pl.ds(off, nrows)])

        _acc_blocks(s, dblock)

    @pl.when(c == 1)
    def _():
        def dblock(off, nrows):
            pltpu.sync_copy(acc_sh.at[pl.ds(off, nrows)],
                            o1_hbm.at[pl.ds(off, nrows)])

        _acc_blocks(s, dblock)


_scatter = pl.kernel(
    _scatter_body,
    out_type=[
        jax.ShapeDtypeStruct((N, H), jnp.float32),
        jax.ShapeDtypeStruct((N, H), jnp.float32),
    ],
    mesh=_MESH,
    scratch_types=[
        pltpu.VMEM((RING_I, CH), jnp.int32),
        pltpu.VMEM((RING_I, CH), jnp.int32),
        pltpu.VMEM((RING_R, CH, H), jnp.float32),
        pltpu.SemaphoreType.DMA,
        pltpu.SemaphoreType.DMA,
        pltpu.SemaphoreType.DMA,
        pltpu.VMEM_SHARED((N, H), jnp.float32),
    ],
)


def _scale_body(o0_ref, o1_ref, deg_ref, out_ref):
    deg = deg_ref[...]
    dis = jnp.where(deg > 0.0, lax.rsqrt(deg), 0.0)
    out_ref[:, :H] = o0_ref[...] * dis
    out_ref[:, H:] = o1_ref[...] * dis


def _scale(o0, o1, deg):
    grid = N // _RB
    return pl.pallas_call(
        _scale_body,
        grid=(grid,),
        in_specs=[
            pl.BlockSpec((_RB, H), lambda i: (i, 0)),
            pl.BlockSpec((_RB, H), lambda i: (i, 0)),
            pl.BlockSpec((_RB, 1), lambda i: (i, 0)),
        ],
        out_specs=pl.BlockSpec((_RB, D_OUT), lambda i: (i, 0)),
        out_shape=jax.ShapeDtypeStruct((N, D_OUT), jnp.float32),
    )(o0, o1, deg)


def kernel(x, edge_index, W):
    ei = edge_index.astype(jnp.int32)
    xw = _matmul(x, W)
    p0, p1 = _hist(ei)
    deg2 = (p0 + p1).reshape(NP, 1)
    y0, y1 = _yscale(xw, deg2)
    o0, o1 = _scatter(y0, y1, ei)
    return _scale(o0, o1, deg2)


# speedup vs baseline: 1.0106x; 1.0014x over previous
import jax
import jax.numpy as jnp
from jax import lax
from jax.experimental import pallas as pl
from jax.experimental.pallas import tpu as pltpu
from jax.experimental.pallas import tpu_sc as plsc

N = 10000
E = 160000
D_IN = 256
D_OUT = 256
H = 128
NP = 10240
STRIPE = NP // 16
CH = 128

_MESH = plsc.VectorSubcoreMesh(core_axis_name="c", subcore_axis_name="s")

def _z16():
    return jnp.zeros((16,), jnp.float32)


def _o16():
    return jnp.ones((16,), jnp.float32)


RING_A = 8
WIN_A = 4


def _hist_body(ei_hbm, p0_hbm, p1_hbm, ridx_v, ones_v, zv, isem, ssem, table_sh):
    c = lax.axis_index("c")
    s = lax.axis_index("s")
    wid = c * 16 + s
    nk = 39 + (wid < 2)

    def zrow(j, carry):
        zv[pl.ds(j * 16, 16)] = _z16()
        return carry

    lax.fori_loop(0, STRIPE // 16, zrow, 0)
    for j in range(CH // 16):
        ones_v[pl.ds(j * 16, 16)] = _o16()
    pltpu.sync_copy(zv, table_sh.at[pl.ds(s * STRIPE, STRIPE)])
    plsc.subcore_barrier()

    def idx_load(i):
        pltpu.async_copy(ei_hbm.at[0, pl.ds((wid + 32 * i) * CH, CH)],
                         ridx_v.at[i % RING_A], isem)

    def idx_wait(i):
        pltpu.make_async_copy(ei_hbm.at[0, pl.ds((wid + 32 * i) * CH, CH)],
                              ridx_v.at[i % RING_A], isem).wait()

    def sc_desc(i):
        return pltpu.make_async_copy(ones_v, table_sh.at[ridx_v.at[i % RING_A]],
                                     ssem)

    for i in range(WIN_A):
        idx_load(i)

    def chunk(i, carry):
        @pl.when(i >= WIN_A)
        def _():
            sc_desc(i - WIN_A).wait()

        @pl.when(i + WIN_A < nk)
        def _():
            idx_load(i + WIN_A)

        idx_wait(i)
        pltpu.async_copy(ones_v, table_sh.at[ridx_v.at[i % RING_A]], ssem,
                         add=True)
        return carry

    lax.fori_loop(0, nk, chunk, 0)

    def drain(j, carry):
        sc_desc(nk - WIN_A + j).wait()
        return carry

    lax.fori_loop(0, WIN_A, drain, 0)
    plsc.subcore_barrier()

    @pl.when(c == 0)
    def _():
        pltpu.sync_copy(table_sh.at[pl.ds(s * STRIPE, STRIPE)],
                        p0_hbm.at[pl.ds(s * STRIPE, STRIPE)])

    @pl.when(c == 1)
    def _():
        pltpu.sync_copy(table_sh.at[pl.ds(s * STRIPE, STRIPE)],
                        p1_hbm.at[pl.ds(s * STRIPE, STRIPE)])


_hist = pl.kernel(
    _hist_body,
    out_type=[
        jax.ShapeDtypeStruct((NP,), jnp.float32),
        jax.ShapeDtypeStruct((NP,), jnp.float32),
    ],
    mesh=_MESH,
    scratch_types=[
        pltpu.VMEM((RING_A, CH), jnp.int32),
        pltpu.VMEM((CH,), jnp.float32),
        pltpu.VMEM((STRIPE,), jnp.float32),
        pltpu.SemaphoreType.DMA,
        pltpu.SemaphoreType.DMA,
        pltpu.VMEM_SHARED((NP,), jnp.float32),
    ],
)


_RB = 1000


def _mm_body(x_ref, w_ref, xw_ref):
    xw_ref[...] = jnp.dot(x_ref[...], w_ref[...],
                          preferred_element_type=jnp.float32)


def _matmul(x, w):
    grid = N // _RB
    return pl.pallas_call(
        _mm_body,
        grid=(grid,),
        in_specs=[
            pl.BlockSpec((_RB, D_IN), lambda i: (i, 0)),
            pl.BlockSpec((D_IN, D_OUT), lambda i: (0, 0)),
        ],
        out_specs=pl.BlockSpec((_RB, D_OUT), lambda i: (i, 0)),
        out_shape=jax.ShapeDtypeStruct((N, D_OUT), jnp.float32),
    )(x, w)


def _yscale_body(xw_ref, deg_ref, y0_ref, y1_ref):
    deg = deg_ref[...]
    dis = jnp.where(deg > 0.0, lax.rsqrt(deg), 0.0)
    y = xw_ref[...] * dis
    y0_ref[...] = y[:, :H]
    y1_ref[...] = y[:, H:]


def _yscale(xw, deg):
    grid = N // _RB
    return pl.pallas_call(
        _yscale_body,
        grid=(grid,),
        in_specs=[
            pl.BlockSpec((_RB, D_OUT), lambda i: (i, 0)),
            pl.BlockSpec((_RB, 1), lambda i: (i, 0)),
        ],
        out_specs=[
            pl.BlockSpec((_RB, H), lambda i: (i, 0)),
            pl.BlockSpec((_RB, H), lambda i: (i, 0)),
        ],
        out_shape=[
            jax.ShapeDtypeStruct((N, H), jnp.float32),
            jax.ShapeDtypeStruct((N, H), jnp.float32),
        ],
    )(xw, deg)


RING_I = 4
RING_R = 3
NZCH = N // CH
ZREM = N % CH


def _acc_blocks(s, copy_block):
    for jj in range(5):
        j = s + 16 * jj

        @pl.when(j < NZCH)
        def _():
            copy_block(j * CH, CH)

        @pl.when(j == NZCH)
        def _():
            copy_block(NZCH * CH, ZREM)


def _scatter_body(y0_hbm, y1_hbm, ei_hbm,
                  o0_hbm, o1_hbm, cidx_v, ridx_v, rows_v,
                  isem, gsem, ssem, acc_sh):
    c = lax.axis_index("c")
    s = lax.axis_index("s")
    nk = 78 + (s < 2)

    def zrow(j, carry):
        for q in range(H // 16):
            rows_v[0, j, pl.ds(q * 16, 16)] = _z16()
        return carry

    lax.fori_loop(0, CH, zrow, 0)

    def zblock(off, nrows):
        pltpu.sync_copy(rows_v.at[0, pl.ds(0, nrows)],
                        acc_sh.at[pl.ds(off, nrows)])

    _acc_blocks(s, zblock)
    plsc.subcore_barrier()

    def idx_load(i):
        e0 = (s + 16 * i) * CH
        pltpu.async_copy(ei_hbm.at[1, pl.ds(e0, CH)], cidx_v.at[i % RING_I], isem)
        pltpu.async_copy(ei_hbm.at[0, pl.ds(e0, CH)], ridx_v.at[i % RING_I], isem)

    def idx_wait(i):
        e0 = (s + 16 * i) * CH
        pltpu.make_async_copy(ei_hbm.at[1, pl.ds(e0, CH)],
                              cidx_v.at[i % RING_I], isem).wait()
        pltpu.make_async_copy(ei_hbm.at[0, pl.ds(e0, CH)],
                              ridx_v.at[i % RING_I], isem).wait()

    def run(y_hbm):
        def gather_go(i):
            pltpu.async_copy(y_hbm.at[cidx_v.at[i % RING_I]],
                             rows_v.at[i % RING_R], gsem)

        def gather_desc(i):
            return pltpu.make_async_copy(y_hbm.at[cidx_v.at[i % RING_I]],
                                         rows_v.at[i % RING_R], gsem)

        def scatter_desc(i):
            return pltpu.make_async_copy(rows_v.at[i % RING_R],
                                         acc_sh.at[ridx_v.at[i % RING_I]], ssem)

        for i in range(3):
            idx_load(i)
        for i in range(2):
            idx_wait(i)
            gather_go(i)

        def chunk(i, carry):
            gather_desc(i).wait()

            @pl.when(i >= 1)
            def _():
                scatter_desc(i - 1).wait()

            @pl.when(i + 3 < nk)
            def _():
                idx_load(i + 3)

            @pl.when(i + 2 < nk)
            def _():
                idx_wait(i + 2)
                gather_go(i + 2)

            pltpu.async_copy(rows_v.at[i % RING_R],
                             acc_sh.at[ridx_v.at[i % RING_I]], ssem, add=True)
            return carry

        lax.fori_loop(0, nk, chunk, 0)
        scatter_desc(nk - 1).wait()

    @pl.when(c == 0)
    def _():
        run(y0_hbm)

    @pl.when(c == 1)
    def _():
        run(y1_hbm)

    plsc.subcore_barrier()

    @pl.when(c == 0)
    def _():
        def dblock(off, nrows):
            pltpu.sync_copy(acc_sh.at[pl.ds(off, nrows)],
                            o0_hbm.at[pl.ds(off, nrows)])

        _acc_blocks(s, dblock)

    @pl.when(c == 1)
    def _():
        def dblock(off, nrows):
            pltpu.sync_copy(acc_sh.at[pl.ds(off, nrows)],
                            o1_hbm.at[pl.ds(off, nrows)])

        _acc_blocks(s, dblock)


_scatter = pl.kernel(
    _scatter_body,
    out_type=[
        jax.ShapeDtypeStruct((N, H), jnp.float32),
        jax.ShapeDtypeStruct((N, H), jnp.float32),
    ],
    mesh=_MESH,
    scratch_types=[
        pltpu.VMEM((RING_I, CH), jnp.int32),
        pltpu.VMEM((RING_I, CH), jnp.int32),
        pltpu.VMEM((RING_R, CH, H), jnp.float32),
        pltpu.SemaphoreType.DMA,
        pltpu.SemaphoreType.DMA,
        pltpu.SemaphoreType.DMA,
        pltpu.VMEM_SHARED((N, H), jnp.float32),
    ],
)


def _scale_body(o0_ref, o1_ref, deg_ref, out_ref):
    deg = deg_ref[...]
    dis = jnp.where(deg > 0.0, lax.rsqrt(deg), 0.0)
    out_ref[:, :H] = o0_ref[...] * dis
    out_ref[:, H:] = o1_ref[...] * dis


def _scale(o0, o1, deg):
    grid = N // _RB
    return pl.pallas_call(
        _scale_body,
        grid=(grid,),
        in_specs=[
            pl.BlockSpec((_RB, H), lambda i: (i, 0)),
            pl.BlockSpec((_RB, H), lambda i: (i, 0)),
            pl.BlockSpec((_RB, 1), lambda i: (i, 0)),
        ],
        out_specs=pl.BlockSpec((_RB, D_OUT), lambda i: (i, 0)),
        out_shape=jax.ShapeDtypeStruct((N, D_OUT), jnp.float32),
    )(o0, o1, deg)


def kernel(x, edge_index, W):
    ei = edge_index.astype(jnp.int32)
    xw = _matmul(x, W)
    p0, p1 = _hist(ei)
    deg2 = (p0 + p1).reshape(NP, 1)
    y0, y1 = _yscale(xw, deg2)
    o0, o1 = _scatter(y0, y1, ei)
    return _scale(o0, o1, deg2)


# speedup vs baseline: 1.0144x; 1.0038x over previous
import jax
import jax.numpy as jnp
from jax import lax
from jax.experimental import pallas as pl
from jax.experimental.pallas import tpu as pltpu
from jax.experimental.pallas import tpu_sc as plsc

N = 10000
E = 160000
D_IN = 256
D_OUT = 256
H = 128
NP = 10240
STRIPE = NP // 16
CH = 128

_MESH = plsc.VectorSubcoreMesh(core_axis_name="c", subcore_axis_name="s")

def _z16():
    return jnp.zeros((16,), jnp.float32)


def _o16():
    return jnp.ones((16,), jnp.float32)


RING_A = 8
WIN_A = 4


def _hist_body(ei_hbm, p0_hbm, p1_hbm, ridx_v, ones_v, zv, isem, ssem, table_sh):
    c = lax.axis_index("c")
    s = lax.axis_index("s")
    wid = c * 16 + s
    nk = 39 + (wid < 2)

    def zrow(j, carry):
        zv[pl.ds(j * 16, 16)] = _z16()
        return carry

    lax.fori_loop(0, STRIPE // 16, zrow, 0)
    for j in range(CH // 16):
        ones_v[pl.ds(j * 16, 16)] = _o16()
    pltpu.sync_copy(zv, table_sh.at[pl.ds(s * STRIPE, STRIPE)])
    plsc.subcore_barrier()

    def idx_load(i):
        pltpu.async_copy(ei_hbm.at[0, pl.ds((wid + 32 * i) * CH, CH)],
                         ridx_v.at[i % RING_A], isem)

    def idx_wait(i):
        pltpu.make_async_copy(ei_hbm.at[0, pl.ds((wid + 32 * i) * CH, CH)],
                              ridx_v.at[i % RING_A], isem).wait()

    def sc_desc(i):
        return pltpu.make_async_copy(ones_v, table_sh.at[ridx_v.at[i % RING_A]],
                                     ssem)

    for i in range(WIN_A):
        idx_load(i)

    def chunk(i, carry):
        @pl.when(i >= WIN_A)
        def _():
            sc_desc(i - WIN_A).wait()

        @pl.when(i + WIN_A < nk)
        def _():
            idx_load(i + WIN_A)

        idx_wait(i)
        pltpu.async_copy(ones_v, table_sh.at[ridx_v.at[i % RING_A]], ssem,
                         add=True)
        return carry

    lax.fori_loop(0, nk, chunk, 0)

    def drain(j, carry):
        sc_desc(nk - WIN_A + j).wait()
        return carry

    lax.fori_loop(0, WIN_A, drain, 0)
    plsc.subcore_barrier()

    @pl.when(c == 0)
    def _():
        pltpu.sync_copy(table_sh.at[pl.ds(s * STRIPE, STRIPE)],
                        p0_hbm.at[pl.ds(s * STRIPE, STRIPE)])

    @pl.when(c == 1)
    def _():
        pltpu.sync_copy(table_sh.at[pl.ds(s * STRIPE, STRIPE)],
                        p1_hbm.at[pl.ds(s * STRIPE, STRIPE)])


_hist = pl.kernel(
    _hist_body,
    out_type=[
        jax.ShapeDtypeStruct((NP,), jnp.float32),
        jax.ShapeDtypeStruct((NP,), jnp.float32),
    ],
    mesh=_MESH,
    scratch_types=[
        pltpu.VMEM((RING_A, CH), jnp.int32),
        pltpu.VMEM((CH,), jnp.float32),
        pltpu.VMEM((STRIPE,), jnp.float32),
        pltpu.SemaphoreType.DMA,
        pltpu.SemaphoreType.DMA,
        pltpu.VMEM_SHARED((NP,), jnp.float32),
    ],
)


_RB = 1000


def _mm_body(x_ref, w_ref, xw_ref):
    xw_ref[...] = jnp.dot(x_ref[...], w_ref[...],
                          preferred_element_type=jnp.float32)


def _matmul(x, w):
    grid = N // _RB
    return pl.pallas_call(
        _mm_body,
        grid=(grid,),
        in_specs=[
            pl.BlockSpec((_RB, D_IN), lambda i: (i, 0)),
            pl.BlockSpec((D_IN, D_OUT), lambda i: (0, 0)),
        ],
        out_specs=pl.BlockSpec((_RB, D_OUT), lambda i: (i, 0)),
        out_shape=jax.ShapeDtypeStruct((N, D_OUT), jnp.float32),
    )(x, w)


def _yscale_body(xw_ref, deg_ref, y0_ref, y1_ref):
    deg = deg_ref[...]
    dis = jnp.where(deg > 0.0, lax.rsqrt(deg), 0.0)
    y = xw_ref[...] * dis
    y0_ref[...] = y[:, :H]
    y1_ref[...] = y[:, H:]


def _yscale(xw, deg):
    grid = N // _RB
    return pl.pallas_call(
        _yscale_body,
        grid=(grid,),
        in_specs=[
            pl.BlockSpec((_RB, D_OUT), lambda i: (i, 0)),
            pl.BlockSpec((_RB, 1), lambda i: (i, 0)),
        ],
        out_specs=[
            pl.BlockSpec((_RB, H), lambda i: (i, 0)),
            pl.BlockSpec((_RB, H), lambda i: (i, 0)),
        ],
        out_shape=[
            jax.ShapeDtypeStruct((N, H), jnp.float32),
            jax.ShapeDtypeStruct((N, H), jnp.float32),
        ],
    )(xw, deg)


RING_I = 4
RING_R = 3
NZCH = N // CH
ZREM = N % CH


def _acc_blocks(s, copy_block):
    for jj in range(5):
        j = s + 16 * jj

        @pl.when(j < NZCH)
        def _():
            copy_block(j * CH, CH)

        @pl.when(j == NZCH)
        def _():
            copy_block(NZCH * CH, ZREM)


def _scatter_body(y0_hbm, y1_hbm, ei_hbm,
                  o0_hbm, o1_hbm, cidx_v, ridx_v, rows_v,
                  isem, gsem, ssem, acc_sh):
    c = lax.axis_index("c")
    s = lax.axis_index("s")
    nk = 78 + (s < 2)

    def idx_load(i):
        e0 = (s + 16 * i) * CH
        pltpu.async_copy(ei_hbm.at[1, pl.ds(e0, CH)], cidx_v.at[i % RING_I], isem)
        pltpu.async_copy(ei_hbm.at[0, pl.ds(e0, CH)], ridx_v.at[i % RING_I], isem)

    for i in range(3):
        idx_load(i)

    def zrow(j, carry):
        for q in range(H // 16):
            rows_v[0, j, pl.ds(q * 16, 16)] = _z16()
        return carry

    lax.fori_loop(0, CH, zrow, 0)

    def zblock(off, nrows):
        pltpu.sync_copy(rows_v.at[0, pl.ds(0, nrows)],
                        acc_sh.at[pl.ds(off, nrows)])

    _acc_blocks(s, zblock)
    plsc.subcore_barrier()

    def idx_wait(i):
        e0 = (s + 16 * i) * CH
        pltpu.make_async_copy(ei_hbm.at[1, pl.ds(e0, CH)],
                              cidx_v.at[i % RING_I], isem).wait()
        pltpu.make_async_copy(ei_hbm.at[0, pl.ds(e0, CH)],
                              ridx_v.at[i % RING_I], isem).wait()

    def run(y_hbm):
        def gather_go(i):
            pltpu.async_copy(y_hbm.at[cidx_v.at[i % RING_I]],
                             rows_v.at[i % RING_R], gsem)

        def gather_desc(i):
            return pltpu.make_async_copy(y_hbm.at[cidx_v.at[i % RING_I]],
                                         rows_v.at[i % RING_R], gsem)

        def scatter_desc(i):
            return pltpu.make_async_copy(rows_v.at[i % RING_R],
                                         acc_sh.at[ridx_v.at[i % RING_I]], ssem)

        for i in range(2):
            idx_wait(i)
            gather_go(i)

        def chunk(i, carry):
            gather_desc(i).wait()

            @pl.when(i >= 1)
            def _():
                scatter_desc(i - 1).wait()

            @pl.when(i + 3 < nk)
            def _():
                idx_load(i + 3)

            @pl.when(i + 2 < nk)
            def _():
                idx_wait(i + 2)
                gather_go(i + 2)

            pltpu.async_copy(rows_v.at[i % RING_R],
                             acc_sh.at[ridx_v.at[i % RING_I]], ssem, add=True)
            return carry

        lax.fori_loop(0, nk, chunk, 0)
        scatter_desc(nk - 1).wait()

    @pl.when(c == 0)
    def _():
        run(y0_hbm)

    @pl.when(c == 1)
    def _():
        run(y1_hbm)

    plsc.subcore_barrier()

    @pl.when(c == 0)
    def _():
        def dblock(off, nrows):
            pltpu.sync_copy(acc_sh.at[pl.ds(off, nrows)],
                            o0_hbm.at[pl.ds(off, nrows)])

        _acc_blocks(s, dblock)

    @pl.when(c == 1)
    def _():
        def dblock(off, nrows):
            pltpu.sync_copy(acc_sh.at[pl.ds(off, nrows)],
                            o1_hbm.at[pl.ds(off, nrows)])

        _acc_blocks(s, dblock)


_scatter = pl.kernel(
    _scatter_body,
    out_type=[
        jax.ShapeDtypeStruct((N, H), jnp.float32),
        jax.ShapeDtypeStruct((N, H), jnp.float32),
    ],
    mesh=_MESH,
    scratch_types=[
        pltpu.VMEM((RING_I, CH), jnp.int32),
        pltpu.VMEM((RING_I, CH), jnp.int32),
        pltpu.VMEM((RING_R, CH, H), jnp.float32),
        pltpu.SemaphoreType.DMA,
        pltpu.SemaphoreType.DMA,
        pltpu.SemaphoreType.DMA,
        pltpu.VMEM_SHARED((N, H), jnp.float32),
    ],
)


def _scale_body(o0_ref, o1_ref, deg_ref, out_ref):
    deg = deg_ref[...]
    dis = jnp.where(deg > 0.0, lax.rsqrt(deg), 0.0)
    out_ref[:, :H] = o0_ref[...] * dis
    out_ref[:, H:] = o1_ref[...] * dis


def _scale(o0, o1, deg):
    grid = N // _RB
    return pl.pallas_call(
        _scale_body,
        grid=(grid,),
        in_specs=[
            pl.BlockSpec((_RB, H), lambda i: (i, 0)),
            pl.BlockSpec((_RB, H), lambda i: (i, 0)),
            pl.BlockSpec((_RB, 1), lambda i: (i, 0)),
        ],
        out_specs=pl.BlockSpec((_RB, D_OUT), lambda i: (i, 0)),
        out_shape=jax.ShapeDtypeStruct((N, D_OUT), jnp.float32),
    )(o0, o1, deg)


def kernel(x, edge_index, W):
    ei = edge_index.astype(jnp.int32)
    xw = _matmul(x, W)
    p0, p1 = _hist(ei)
    deg2 = (p0 + p1).reshape(NP, 1)
    y0, y1 = _yscale(xw, deg2)
    o0, o1 = _scatter(y0, y1, ei)
    return _scale(o0, o1, deg2)
